# baseline (speedup 1.0000x reference)
import functools

import jax
import jax.numpy as jnp
from jax import lax
from jax.experimental import pallas as pl
from jax.experimental.pallas import tpu as pltpu
from jax.experimental.pallas import tpu_sc as plsc

NC = 2
NS = 16
CW = 32
RING = 8
LOOK = 6
SLABC = 32
ZB = 16


def _sc_aggregate(x2, col2, row2, vals, *, n, d, ep):
    e = ep * NS
    nch = ep // CW
    nslab = nch // SLABC
    sedge = SLABC * CW
    stripe = n // NS // 8 * 8
    last_stripe = n - stripe * (NS - 1)

    mesh = plsc.VectorSubcoreMesh(core_axis_name="c", subcore_axis_name="s")

    @functools.partial(
        pl.kernel,
        out_type=jax.ShapeDtypeStruct((NC * n, d), jnp.float32),
        mesh=mesh,
        scratch_types=[
            pltpu.VMEM((2, SLABC * CW // 128, 128), jnp.int32),
            pltpu.VMEM((2, SLABC, CW), jnp.int32),
            pltpu.VMEM((2, SLABC * CW), jnp.float32),
            pltpu.VMEM((RING, CW, d), jnp.float32),
            pltpu.VMEM_SHARED((n, d), jnp.float32),
            pltpu.SemaphoreType.DMA,
            pltpu.SemaphoreType.DMA,
            pltpu.SemaphoreType.DMA,
        ],
    )
    def body(x_hbm, col_hbm, row_hbm, val_hbm, out_hbm,
             colv, rowv, valv, bufs, agg, gsem, ssem, stsem):
        c = lax.axis_index("c")
        s = lax.axis_index("s")

        ebase = c * e + s * ep
        cbase = pl.multiple_of(ebase // 128, 8)
        rbase = pl.multiple_of(ebase // CW, 8)

        def gidx(g):
            m = g // SLABC
            q = g % SLABC
            return colv.at[m % 2, q // 4, pl.ds(q % 4 * CW, CW)]

        def ridx(g):
            return rowv.at[(g // SLABC) % 2, g % SLABC]

        def stage(m, sync=False):
            sl = m % 2
            srcs_dsts = [
                (col_hbm.at[pl.ds(pl.multiple_of(
                    cbase + m * (sedge // 128), 8), sedge // 128)],
                 colv.at[sl]),
                (row_hbm.at[pl.ds(pl.multiple_of(rbase + m * SLABC, 8),
                                  SLABC)],
                 rowv.at[sl]),
                (val_hbm.at[pl.ds(ebase + m * sedge, sedge)], valv.at[sl]),
            ]
            for src, dst in srcs_dsts:
                if sync:
                    pltpu.sync_copy(src, dst)
                else:
                    pltpu.async_copy(src, dst, stsem)

        def stage_wait(m):
            sl = m % 2
            pltpu.make_async_copy(
                col_hbm.at[pl.ds(cbase, sedge // 128)], colv.at[sl],
                stsem).wait()
            pltpu.make_async_copy(
                row_hbm.at[pl.ds(rbase, SLABC)], rowv.at[sl], stsem).wait()
            pltpu.make_async_copy(
                val_hbm.at[pl.ds(ebase, sedge)], valv.at[sl], stsem).wait()

        stage(0, sync=True)

        def bfill(r, carry):
            for u in range(d // 16):
                bufs[RING - 1, r, pl.ds(u * 16, 16)] = jnp.zeros(
                    (16,), jnp.float32)
            return carry
        lax.fori_loop(0, CW, bfill, 0)

        sbase = pl.multiple_of(s * stripe, 8)
        nblk = jnp.where(s == NS - 1, last_stripe // ZB, stripe // ZB)

        def zcopy(t, carry):
            off = pl.multiple_of(sbase + t * ZB, 8)
            pltpu.sync_copy(bufs.at[RING - 1, pl.ds(0, ZB)],
                            agg.at[pl.ds(off, ZB)])
            return carry
        lax.fori_loop(0, nblk, zcopy, 0)
        plsc.subcore_barrier()

        for g0 in range(LOOK):
            pltpu.async_copy(
                x_hbm.at[colv.at[0, g0 // 4, pl.ds(g0 % 4 * CW, CW)]],
                bufs.at[g0 % RING], gsem)
        pltpu.async_copy(bufs.at[RING - 1], agg.at[rowv.at[0, 0]], ssem,
                         add=True)

        def chunk_body(g, wait_s=True, issue=True):
            p = g % RING
            pltpu.make_async_copy(x_hbm.at[gidx(g)], bufs.at[p],
                                  gsem).wait()
            if wait_s:
                pltpu.make_async_copy(bufs.at[p], agg.at[ridx(g)],
                                      ssem).wait()
            if issue:
                gn = g + LOOK
                pltpu.async_copy(x_hbm.at[gidx(gn)], bufs.at[gn % RING],
                                 gsem)
            sl = (g // SLABC) % 2
            vq = g % SLABC * CW

            def edge_body(ei, ecarry):
                eib = ei // 16 * 16
                grp = valv[sl, pl.ds(vq + eib, 16)]
                v16 = grp.at[jnp.full((16,), ei - eib, jnp.int32)].get(
                    mode="promise_in_bounds")
                for u in range(d // 16):
                    slc = (p, ei, pl.ds(u * 16, 16))
                    bufs[slc] = bufs[slc] * v16
                return ecarry
            lax.fori_loop(0, CW, edge_body, 0)
            pltpu.async_copy(bufs.at[p], agg.at[ridx(g)], ssem, add=True)

        def seg(lo, hi):
            def sbody(g, carry):
                chunk_body(g)
                return carry
            lax.fori_loop(lo, hi, sbody, 0)

        chunk_body(0, wait_s=False)
        chunk_body(1)
        stage(1)
        seg(2, SLABC - 8)
        stage_wait(1)
        seg(SLABC - 8, SLABC)

        def slab_body(t, carry):
            lo = t * SLABC
            seg(lo, lo + 2)
            stage(t + 1)
            seg(lo + 2, lo + SLABC - 8)
            stage_wait(t + 1)
            seg(lo + SLABC - 8, lo + SLABC)
            return carry
        lax.fori_loop(1, nslab - 1, slab_body, 0)

        lo = (nslab - 1) * SLABC
        seg(lo, lo + SLABC - LOOK)

        def tail_body(g, carry):
            chunk_body(g, issue=False)
            return carry
        lax.fori_loop(lo + SLABC - LOOK, lo + SLABC, tail_body, 0)

        pltpu.make_async_copy(bufs.at[0], agg.at[rowv.at[0, 0]],
                              ssem).wait()
        pltpu.make_async_copy(bufs.at[0], agg.at[rowv.at[0, 0]],
                              ssem).wait()
        plsc.subcore_barrier()

        def drain(t, carry):
            off = pl.multiple_of(sbase + t * ZB, 8)
            pltpu.sync_copy(
                agg.at[pl.ds(off, ZB)],
                out_hbm.at[pl.ds(pl.multiple_of(c * n + sbase + t * ZB, 8),
                                 ZB)],
            )
            return carry
        lax.fori_loop(0, nblk, drain, 0)

    return body(x2, col2, row2, vals)


def _mm_relu_kernel(a_ref, w_ref, o_ref):
    o_ref[...] = jnp.maximum(
        jnp.dot(a_ref[...], w_ref[...], preferred_element_type=jnp.float32),
        0.0,
    )


def kernel(x, adj_indices, adj_values, W):
    b, n, d = x.shape
    e = adj_indices.shape[1]
    dout = W.shape[1]

    row = adj_indices[..., 0].astype(jnp.int32)
    col = adj_indices[..., 1].astype(jnp.int32)
    align = NS * SLABC * CW
    e_pad = -(-e // align) * align
    pad = e_pad - e
    if pad:
        zi = jnp.zeros((b, pad), jnp.int32)
        row = jnp.concatenate([row, zi], axis=1)
        col = jnp.concatenate([col, zi], axis=1)
        adj_values = jnp.concatenate(
            [adj_values, jnp.zeros((b, pad), adj_values.dtype)], axis=1)
    colg = col + (jnp.arange(b, dtype=jnp.int32) * n)[:, None]
    col2 = colg.reshape(b * e_pad // 128, 128)
    row2 = row.reshape(b * e_pad // CW, CW)
    vals = adj_values.reshape(b * e_pad)
    x2 = x.reshape(b * n, d)

    agg = _sc_aggregate(x2, col2, row2, vals, n=n, d=d, ep=e_pad // NS)

    rows_total = b * n
    blk = 2000
    out = pl.pallas_call(
        _mm_relu_kernel,
        grid=(rows_total // blk,),
        in_specs=[
            pl.BlockSpec((blk, d), lambda i: (i, 0)),
            pl.BlockSpec((d, dout), lambda i: (0, 0)),
        ],
        out_specs=pl.BlockSpec((blk, dout), lambda i: (i, 0)),
        out_shape=jax.ShapeDtypeStruct((rows_total, dout), jnp.float32),
    )(agg, W)
    return out.reshape(b, n, dout)
